# baseline (speedup 1.0000x reference)
import functools

import jax
import jax.numpy as jnp
from jax import lax
from jax.experimental import pallas as pl
from jax.experimental.pallas import tpu as pltpu
from jax.experimental.pallas import tpu_sc as plsc

N = 10000
E = 320000
D = 128
L = 5
G = 256
T = 128

CH = 64
NCHUNKS = E // CH
NW = 32
SUBCORES = 16
ROWS_PER_TILE = 624
TAIL_BASE = SUBCORES * ROWS_PER_TILE
TAIL_ROWS = N - TAIL_BASE

K = 157
NBUF = 3
IR = 6
GROUPS = (K + IR - 1) // IR


def _edge_inner(h_hbm, ea_hbm, src_hbm, dst_hbm, out_hbm,
                srcix, dstix, rows, eab, agg_sh, sp, si, sg, se, ss):
    c = lax.axis_index("c")
    s = lax.axis_index("s")
    w = s * 2 + c
    ck0 = K * w
    n = jnp.minimum(K, NCHUNKS - ck0)

    def _issue_src(p, qp):
        e0 = (ck0 + 2 * p) * CH
        pltpu.async_copy(src_hbm.at[pl.ds(e0, 2 * CH)], srcix[qp], sp[qp])

    def _wait_src(p, qp):
        e0 = (ck0 + 2 * p) * CH
        pltpu.make_async_copy(src_hbm.at[pl.ds(e0, 2 * CH)], srcix[qp],
                              sp[qp]).wait()

    def _issue_dst(m, q):
        e0 = (ck0 + m) * CH
        pltpu.async_copy(dst_hbm.at[pl.ds(e0, CH)], dstix[q], si[q])

    def _wait_dst(m, q):
        e0 = (ck0 + m) * CH
        pltpu.make_async_copy(dst_hbm.at[pl.ds(e0, CH)], dstix[q], si[q]).wait()

    def _issue_data(m, b, qp, half):
        pltpu.async_copy(h_hbm.at[srcix[qp].at[pl.ds(half * CH, CH)]],
                         rows[b], sg[b])
        pltpu.async_copy(ea_hbm.at[pl.ds((ck0 + m) * CH, CH)], eab[b], se[b])

    for pp in range(3):
        _issue_src(pp, pp)
    for mm in range(4):
        _issue_dst(mm, mm)

    def _zrow(r, carry):
        for j in range(D // 16):
            rows[0][r, pl.ds(j * 16, 16)] = jnp.zeros((16,), jnp.float32)
        return carry
    lax.fori_loop(0, CH, _zrow, None)
    base = s * ROWS_PER_TILE
    off = 0
    while off < ROWS_PER_TILE:
        cnt = min(CH, ROWS_PER_TILE - off)
        pltpu.sync_copy(rows[0].at[pl.ds(0, cnt)], agg_sh.at[pl.ds(base + off, cnt)])
        off += cnt

    @pl.when(s == SUBCORES - 1)
    def _():
        pltpu.sync_copy(rows[0].at[pl.ds(0, TAIL_ROWS)],
                        agg_sh.at[pl.ds(TAIL_BASE, TAIL_ROWS)])

    _wait_src(0, 0)
    for mm in range(2):
        _wait_dst(mm, mm)
        _issue_data(mm, mm, 0, mm)
    plsc.subcore_barrier()

    def _group(g, carry):
        for u in range(IR):
            m = g * IR + u
            b = u % NBUF
            q = u
            qp = u // 2
            b2 = (b + 2) % NBUF
            q2 = (u + 2) % IR
            qp2 = ((u + 2) // 2) % 3

            @pl.when(m < n)
            def _():
                pltpu.make_async_copy(
                    h_hbm.at[srcix[qp].at[pl.ds((u % 2) * CH, CH)]],
                    rows[b], sg[b]).wait()
                pltpu.make_async_copy(
                    ea_hbm.at[pl.ds((ck0 + m) * CH, CH)], eab[b], se[b]).wait()

                def _row(r4, carry2):
                    r = r4 * 4
                    for dr in range(4):
                        for j in range(D // 16):
                            sl = pl.ds(j * 16, 16)
                            rows[b][r + dr, sl] = jnp.maximum(
                                rows[b][r + dr, sl] + eab[b][r + dr, sl], 0.0)
                    return carry2
                lax.fori_loop(0, CH // 4, _row, None)
                pltpu.async_copy(rows[b], agg_sh.at[dstix[q]], ss[b], add=True)

                @pl.when(m + 2 < n)
                def _():
                    @pl.when(m >= 1)
                    def _():
                        pltpu.make_async_copy(
                            rows[b2], agg_sh.at[dstix[q2]], ss[b2]).wait()
                    if u % 2 == 0:
                        _wait_src((m + 2) // 2, qp2)
                    _wait_dst(m + 2, q2)
                    _issue_data(m + 2, b2, qp2, u % 2)

                    @pl.when(m + 4 < n)
                    def _():
                        _issue_dst(m + 4, (u + 4) % IR)
                if u % 2 == 1:
                    @pl.when(m + 5 < n)
                    def _():
                        _issue_src((m + 5) // 2, ((u + 5) // 2) % 3)
        return carry

    lax.fori_loop(0, GROUPS, _group, None)

    for b in range(NBUF):
        pltpu.make_async_copy(rows[b], agg_sh.at[dstix[0]], ss[b]).wait()
    plsc.subcore_barrier()

    pltpu.sync_copy(agg_sh.at[pl.ds(base, ROWS_PER_TILE)],
                    out_hbm.at[pl.ds(c * N + base, ROWS_PER_TILE)])

    @pl.when(s == SUBCORES - 1)
    def _():
        pltpu.sync_copy(agg_sh.at[pl.ds(TAIL_BASE, TAIL_ROWS)],
                        out_hbm.at[pl.ds(c * N + TAIL_BASE, TAIL_ROWS)])


_edge_call = functools.partial(
    pl.kernel,
    out_type=jax.ShapeDtypeStruct((2 * N, D), jnp.float32),
    mesh=plsc.VectorSubcoreMesh(core_axis_name="c", subcore_axis_name="s"),
    scratch_types=[
        tuple(pltpu.VMEM((2 * CH,), jnp.int32) for _ in range(3)),
        tuple(pltpu.VMEM((CH,), jnp.int32) for _ in range(IR)),
        tuple(pltpu.VMEM((CH, D), jnp.float32) for _ in range(NBUF)),
        tuple(pltpu.VMEM((CH, D), jnp.float32) for _ in range(NBUF)),
        pltpu.VMEM_SHARED((N, D), jnp.float32),
        tuple(pltpu.SemaphoreType.DMA for _ in range(3)),
        tuple(pltpu.SemaphoreType.DMA for _ in range(IR)),
        tuple(pltpu.SemaphoreType.DMA for _ in range(NBUF)),
        tuple(pltpu.SemaphoreType.DMA for _ in range(NBUF)),
        tuple(pltpu.SemaphoreType.DMA for _ in range(NBUF)),
    ],
)(_edge_inner)


BN = 1000


def _mlp_body(relu_last, scale_ref, h_ref, agg_ref, w1_ref, b1_ref, g1_ref,
              bb1_ref, w2_ref, b2_ref, g_ref, bb_ref, o_ref):
    h = h_ref[...]
    z = h * scale_ref[0, 0] + agg_ref[0] + agg_ref[1]
    z = jnp.dot(z, w1_ref[...], preferred_element_type=jnp.float32) + b1_ref[...]
    z = jnp.maximum(z * g1_ref[...] + bb1_ref[...], 0.0)
    z = jnp.dot(z, w2_ref[...], preferred_element_type=jnp.float32) + b2_ref[...]
    z = z * g_ref[...] + bb_ref[...]
    if relu_last:
        z = jnp.maximum(z, 0.0)
    o_ref[...] = z + h


def _make_mlp(relu_last):
    return pl.pallas_call(
        functools.partial(_mlp_body, relu_last),
        grid=(N // BN,),
        in_specs=[
            pl.BlockSpec(memory_space=pltpu.SMEM),
            pl.BlockSpec((BN, D), lambda i: (i, 0)),
            pl.BlockSpec((2, BN, D), lambda i: (0, i, 0)),
            pl.BlockSpec((D, 2 * D), lambda i: (0, 0)),
            pl.BlockSpec((1, 2 * D), lambda i: (0, 0)),
            pl.BlockSpec((1, 2 * D), lambda i: (0, 0)),
            pl.BlockSpec((1, 2 * D), lambda i: (0, 0)),
            pl.BlockSpec((2 * D, D), lambda i: (0, 0)),
            pl.BlockSpec((1, D), lambda i: (0, 0)),
            pl.BlockSpec((1, D), lambda i: (0, 0)),
            pl.BlockSpec((1, D), lambda i: (0, 0)),
        ],
        out_specs=pl.BlockSpec((BN, D), lambda i: (i, 0)),
        out_shape=jax.ShapeDtypeStruct((N, D), jnp.float32),
    )


_mlp_relu = _make_mlp(True)
_mlp_last = _make_mlp(False)


def _pool_body(batch_ref, h_ref, pw1_ref, pb1_ref, pg_ref, pbb_ref, pw2_ref,
               pb2_ref, pred_ref, hg_ref):
    i = pl.program_id(0)
    b = batch_ref[0, 0, :]
    oh = (b[:, None] == lax.broadcasted_iota(jnp.int32, (BN, G), 1)).astype(jnp.float32)
    part = lax.dot_general(oh, h_ref[...], (((0,), (0,)), ((), ())),
                           preferred_element_type=jnp.float32)

    @pl.when(i == 0)
    def _():
        hg_ref[...] = jnp.zeros_like(hg_ref)

    hg_ref[...] += part

    @pl.when(i == pl.num_programs(0) - 1)
    def _():
        hg = hg_ref[...]
        hid = jnp.dot(hg, pw1_ref[...], preferred_element_type=jnp.float32)
        hid = jnp.maximum((hid + pb1_ref[...]) * pg_ref[...] + pbb_ref[...], 0.0)
        pred_ref[...] = (jnp.dot(hid, pw2_ref[...],
                                 preferred_element_type=jnp.float32) + pb2_ref[...])


_pool_call = pl.pallas_call(
    _pool_body,
    grid=(N // BN,),
    in_specs=[
        pl.BlockSpec((1, 1, BN), lambda i: (i, 0, 0)),
        pl.BlockSpec((BN, D), lambda i: (i, 0)),
        pl.BlockSpec((D, 2 * D), lambda i: (0, 0)),
        pl.BlockSpec((1, 2 * D), lambda i: (0, 0)),
        pl.BlockSpec((1, 2 * D), lambda i: (0, 0)),
        pl.BlockSpec((1, 2 * D), lambda i: (0, 0)),
        pl.BlockSpec((2 * D, T), lambda i: (0, 0)),
        pl.BlockSpec((1, T), lambda i: (0, 0)),
    ],
    out_specs=[
        pl.BlockSpec((G, T), lambda i: (0, 0)),
        pl.BlockSpec((G, D), lambda i: (0, 0)),
    ],
    out_shape=[
        jax.ShapeDtypeStruct((G, T), jnp.float32),
        jax.ShapeDtypeStruct((G, D), jnp.float32),
    ],
)


def kernel(x, edge_index, edge_attr, batch, W1, b1, bn1_g, bn1_b, W2, b2, eps,
           bn_g, bn_b, pW1, pb1, pbn_g, pbn_b, pW2, pb2):
    src = jnp.concatenate([edge_index[0], jnp.zeros((CH,), jnp.int32)])
    dst = edge_index[1]
    h = x
    for l in range(L):
        agg2 = _edge_call(h, edge_attr, src, dst).reshape(2, N, D)
        scale = (1.0 + eps[l]).reshape(1, 1)
        mlp = _mlp_relu if l < L - 1 else _mlp_last
        h = mlp(scale, h, agg2, W1[l], b1[l].reshape(1, -1),
                bn1_g[l].reshape(1, -1), bn1_b[l].reshape(1, -1), W2[l],
                b2[l].reshape(1, -1), bn_g[l].reshape(1, -1),
                bn_b[l].reshape(1, -1))
    pred, hg = _pool_call(batch.reshape(N // BN, 1, BN), h, pW1,
                          pb1.reshape(1, -1), pbn_g.reshape(1, -1),
                          pbn_b.reshape(1, -1), pW2, pb2.reshape(1, -1))
    return (pred, hg)


# speedup vs baseline: 1.0023x; 1.0023x over previous
import functools

import jax
import jax.numpy as jnp
from jax import lax
from jax.experimental import pallas as pl
from jax.experimental.pallas import tpu as pltpu
from jax.experimental.pallas import tpu_sc as plsc

N = 10000
E = 320000
D = 128
L = 5
G = 256
T = 128

CH = 64
NCHUNKS = E // CH
NW = 32
SUBCORES = 16
ROWS_PER_TILE = 624
TAIL_BASE = SUBCORES * ROWS_PER_TILE
TAIL_ROWS = N - TAIL_BASE

K = 157
NBUF = 3
IR = 6
GROUPS = (K + IR - 1) // IR


def _edge_inner(h_hbm, ea_hbm, src_hbm, dst_hbm, out_hbm,
                srcix, dstix, rows, eab, agg_sh, sp, si, sg, se, ss):
    c = lax.axis_index("c")
    s = lax.axis_index("s")
    w = s * 2 + c
    ck0 = K * w
    n = jnp.minimum(K, NCHUNKS - ck0)

    def _issue_src(p, qp):
        e0 = (ck0 + 2 * p) * CH
        pltpu.async_copy(src_hbm.at[pl.ds(e0, 2 * CH)], srcix[qp], sp[qp])

    def _wait_src(p, qp):
        e0 = (ck0 + 2 * p) * CH
        pltpu.make_async_copy(src_hbm.at[pl.ds(e0, 2 * CH)], srcix[qp],
                              sp[qp]).wait()

    def _issue_dst(m, q):
        e0 = (ck0 + m) * CH
        pltpu.async_copy(dst_hbm.at[pl.ds(e0, CH)], dstix[q], si[q])

    def _wait_dst(m, q):
        e0 = (ck0 + m) * CH
        pltpu.make_async_copy(dst_hbm.at[pl.ds(e0, CH)], dstix[q], si[q]).wait()

    def _issue_data(m, b, qp, half):
        pltpu.async_copy(h_hbm.at[srcix[qp].at[pl.ds(half * CH, CH)]],
                         rows[b], sg[b])
        pltpu.async_copy(ea_hbm.at[pl.ds((ck0 + m) * CH, CH)], eab[b], se[b])

    for pp in range(3):
        _issue_src(pp, pp)
    for mm in range(4):
        _issue_dst(mm, mm)

    def _zrow(r, carry):
        for j in range(D // 16):
            rows[0][r, pl.ds(j * 16, 16)] = jnp.zeros((16,), jnp.float32)
        return carry
    lax.fori_loop(0, CH, _zrow, None)
    base = s * ROWS_PER_TILE
    zcopies = []
    off = 0
    while off < ROWS_PER_TILE:
        cnt = min(CH, ROWS_PER_TILE - off)
        zcopies.append((rows[0].at[pl.ds(0, cnt)],
                        agg_sh.at[pl.ds(base + off, cnt)]))
        off += cnt
    for zsrc, zdst in zcopies:
        pltpu.async_copy(zsrc, zdst, ss[0])
    for zsrc, zdst in zcopies:
        pltpu.make_async_copy(zsrc, zdst, ss[0]).wait()

    @pl.when(s == SUBCORES - 1)
    def _():
        pltpu.sync_copy(rows[0].at[pl.ds(0, TAIL_ROWS)],
                        agg_sh.at[pl.ds(TAIL_BASE, TAIL_ROWS)])

    _wait_src(0, 0)
    for mm in range(2):
        _wait_dst(mm, mm)
        _issue_data(mm, mm, 0, mm)
    plsc.subcore_barrier()

    def _group(g, carry):
        for u in range(IR):
            m = g * IR + u
            b = u % NBUF
            q = u
            qp = u // 2
            b2 = (b + 2) % NBUF
            q2 = (u + 2) % IR
            qp2 = ((u + 2) // 2) % 3

            @pl.when(m < n)
            def _():
                pltpu.make_async_copy(
                    h_hbm.at[srcix[qp].at[pl.ds((u % 2) * CH, CH)]],
                    rows[b], sg[b]).wait()
                pltpu.make_async_copy(
                    ea_hbm.at[pl.ds((ck0 + m) * CH, CH)], eab[b], se[b]).wait()

                def _row(r4, carry2):
                    r = r4 * 4
                    for dr in range(4):
                        for j in range(D // 16):
                            sl = pl.ds(j * 16, 16)
                            rows[b][r + dr, sl] = jnp.maximum(
                                rows[b][r + dr, sl] + eab[b][r + dr, sl], 0.0)
                    return carry2
                lax.fori_loop(0, CH // 4, _row, None)
                pltpu.async_copy(rows[b], agg_sh.at[dstix[q]], ss[b], add=True)

                @pl.when(m + 2 < n)
                def _():
                    @pl.when(m >= 1)
                    def _():
                        pltpu.make_async_copy(
                            rows[b2], agg_sh.at[dstix[q2]], ss[b2]).wait()
                    if u % 2 == 0:
                        _wait_src((m + 2) // 2, qp2)
                    _wait_dst(m + 2, q2)
                    _issue_data(m + 2, b2, qp2, u % 2)

                    @pl.when(m + 4 < n)
                    def _():
                        _issue_dst(m + 4, (u + 4) % IR)
                if u % 2 == 1:
                    @pl.when(m + 5 < n)
                    def _():
                        _issue_src((m + 5) // 2, ((u + 5) // 2) % 3)
        return carry

    lax.fori_loop(0, GROUPS, _group, None)

    for b in range(NBUF):
        pltpu.make_async_copy(rows[b], agg_sh.at[dstix[0]], ss[b]).wait()
    plsc.subcore_barrier()

    pltpu.sync_copy(agg_sh.at[pl.ds(base, ROWS_PER_TILE)],
                    out_hbm.at[pl.ds(c * N + base, ROWS_PER_TILE)])

    @pl.when(s == SUBCORES - 1)
    def _():
        pltpu.sync_copy(agg_sh.at[pl.ds(TAIL_BASE, TAIL_ROWS)],
                        out_hbm.at[pl.ds(c * N + TAIL_BASE, TAIL_ROWS)])


_edge_call = functools.partial(
    pl.kernel,
    out_type=jax.ShapeDtypeStruct((2 * N, D), jnp.float32),
    mesh=plsc.VectorSubcoreMesh(core_axis_name="c", subcore_axis_name="s"),
    scratch_types=[
        tuple(pltpu.VMEM((2 * CH,), jnp.int32) for _ in range(3)),
        tuple(pltpu.VMEM((CH,), jnp.int32) for _ in range(IR)),
        tuple(pltpu.VMEM((CH, D), jnp.float32) for _ in range(NBUF)),
        tuple(pltpu.VMEM((CH, D), jnp.float32) for _ in range(NBUF)),
        pltpu.VMEM_SHARED((N, D), jnp.float32),
        tuple(pltpu.SemaphoreType.DMA for _ in range(3)),
        tuple(pltpu.SemaphoreType.DMA for _ in range(IR)),
        tuple(pltpu.SemaphoreType.DMA for _ in range(NBUF)),
        tuple(pltpu.SemaphoreType.DMA for _ in range(NBUF)),
        tuple(pltpu.SemaphoreType.DMA for _ in range(NBUF)),
    ],
)(_edge_inner)


BN = 1000


def _mlp_body(relu_last, scale_ref, h_ref, agg_ref, w1_ref, b1_ref, g1_ref,
              bb1_ref, w2_ref, b2_ref, g_ref, bb_ref, o_ref):
    h = h_ref[...]
    z = h * scale_ref[0, 0] + agg_ref[0] + agg_ref[1]
    z = jnp.dot(z, w1_ref[...], preferred_element_type=jnp.float32) + b1_ref[...]
    z = jnp.maximum(z * g1_ref[...] + bb1_ref[...], 0.0)
    z = jnp.dot(z, w2_ref[...], preferred_element_type=jnp.float32) + b2_ref[...]
    z = z * g_ref[...] + bb_ref[...]
    if relu_last:
        z = jnp.maximum(z, 0.0)
    o_ref[...] = z + h


def _make_mlp(relu_last):
    return pl.pallas_call(
        functools.partial(_mlp_body, relu_last),
        grid=(N // BN,),
        in_specs=[
            pl.BlockSpec(memory_space=pltpu.SMEM),
            pl.BlockSpec((BN, D), lambda i: (i, 0)),
            pl.BlockSpec((2, BN, D), lambda i: (0, i, 0)),
            pl.BlockSpec((D, 2 * D), lambda i: (0, 0)),
            pl.BlockSpec((1, 2 * D), lambda i: (0, 0)),
            pl.BlockSpec((1, 2 * D), lambda i: (0, 0)),
            pl.BlockSpec((1, 2 * D), lambda i: (0, 0)),
            pl.BlockSpec((2 * D, D), lambda i: (0, 0)),
            pl.BlockSpec((1, D), lambda i: (0, 0)),
            pl.BlockSpec((1, D), lambda i: (0, 0)),
            pl.BlockSpec((1, D), lambda i: (0, 0)),
        ],
        out_specs=pl.BlockSpec((BN, D), lambda i: (i, 0)),
        out_shape=jax.ShapeDtypeStruct((N, D), jnp.float32),
    )


_mlp_relu = _make_mlp(True)
_mlp_last = _make_mlp(False)


def _pool_body(batch_ref, h_ref, pw1_ref, pb1_ref, pg_ref, pbb_ref, pw2_ref,
               pb2_ref, pred_ref, hg_ref):
    i = pl.program_id(0)
    b = batch_ref[0, 0, :]
    oh = (b[:, None] == lax.broadcasted_iota(jnp.int32, (BN, G), 1)).astype(jnp.float32)
    part = lax.dot_general(oh, h_ref[...], (((0,), (0,)), ((), ())),
                           preferred_element_type=jnp.float32)

    @pl.when(i == 0)
    def _():
        hg_ref[...] = jnp.zeros_like(hg_ref)

    hg_ref[...] += part

    @pl.when(i == pl.num_programs(0) - 1)
    def _():
        hg = hg_ref[...]
        hid = jnp.dot(hg, pw1_ref[...], preferred_element_type=jnp.float32)
        hid = jnp.maximum((hid + pb1_ref[...]) * pg_ref[...] + pbb_ref[...], 0.0)
        pred_ref[...] = (jnp.dot(hid, pw2_ref[...],
                                 preferred_element_type=jnp.float32) + pb2_ref[...])


_pool_call = pl.pallas_call(
    _pool_body,
    grid=(N // BN,),
    in_specs=[
        pl.BlockSpec((1, 1, BN), lambda i: (i, 0, 0)),
        pl.BlockSpec((BN, D), lambda i: (i, 0)),
        pl.BlockSpec((D, 2 * D), lambda i: (0, 0)),
        pl.BlockSpec((1, 2 * D), lambda i: (0, 0)),
        pl.BlockSpec((1, 2 * D), lambda i: (0, 0)),
        pl.BlockSpec((1, 2 * D), lambda i: (0, 0)),
        pl.BlockSpec((2 * D, T), lambda i: (0, 0)),
        pl.BlockSpec((1, T), lambda i: (0, 0)),
    ],
    out_specs=[
        pl.BlockSpec((G, T), lambda i: (0, 0)),
        pl.BlockSpec((G, D), lambda i: (0, 0)),
    ],
    out_shape=[
        jax.ShapeDtypeStruct((G, T), jnp.float32),
        jax.ShapeDtypeStruct((G, D), jnp.float32),
    ],
)


def kernel(x, edge_index, edge_attr, batch, W1, b1, bn1_g, bn1_b, W2, b2, eps,
           bn_g, bn_b, pW1, pb1, pbn_g, pbn_b, pW2, pb2):
    src = jnp.concatenate([edge_index[0], jnp.zeros((CH,), jnp.int32)])
    dst = edge_index[1]
    h = x
    for l in range(L):
        agg2 = _edge_call(h, edge_attr, src, dst).reshape(2, N, D)
        scale = (1.0 + eps[l]).reshape(1, 1)
        mlp = _mlp_relu if l < L - 1 else _mlp_last
        h = mlp(scale, h, agg2, W1[l], b1[l].reshape(1, -1),
                bn1_g[l].reshape(1, -1), bn1_b[l].reshape(1, -1), W2[l],
                b2[l].reshape(1, -1), bn_g[l].reshape(1, -1),
                bn_b[l].reshape(1, -1))
    pred, hg = _pool_call(batch.reshape(N // BN, 1, BN), h, pW1,
                          pb1.reshape(1, -1), pbn_g.reshape(1, -1),
                          pbn_b.reshape(1, -1), pW2, pb2.reshape(1, -1))
    return (pred, hg)
